# baseline (speedup 1.0000x reference)
import functools

import jax
import jax.numpy as jnp
from jax import lax
from jax.experimental import pallas as pl
from jax.experimental.pallas import tpu as pltpu
from jax.experimental.pallas import tpu_sc as plsc

NC = 2
NS = 16
NW = NC * NS
NB = 2
GROUPS = (0, 16, 32, 34)


SEG = 128


@functools.lru_cache(maxsize=None)
def _build(bsz, hist, d, n_sup):
    assert bsz % (NW * SEG) == 0 and SEG % NB == 0
    rows_per_w = bsz // NW
    n_segs = rows_per_w // SEG
    n_groups = SEG // NB
    mesh = plsc.VectorSubcoreMesh(core_axis_name="c", subcore_axis_name="s")

    @functools.partial(
        pl.kernel,
        mesh=mesh,
        compiler_params=pltpu.CompilerParams(needs_layout_passes=False),
        out_type=jax.ShapeDtypeStruct((bsz, hist, d), jnp.float32),
        scratch_types=[
            pltpu.VMEM((SEG, hist), jnp.int32),
            pltpu.VMEM((SEG, hist), jnp.int32),
            *[pltpu.VMEM((hist, 4 * d), jnp.float32) for _ in range(NB)],
            *[pltpu.VMEM((hist, d), jnp.float32) for _ in range(NB)],
            *[pltpu.SemaphoreType.DMA for _ in range(2 * NB)],
        ],
    )
    def body(sidx_hbm, ofs_hbm, table_hbm, out_hbm, sidx_v, ofs_v, *rest):
        sup_bufs = rest[:NB]
        out_bufs = rest[NB:2 * NB]
        gsems = rest[2 * NB:3 * NB]
        ssems = rest[3 * NB:]
        wid = lax.axis_index("s") * NC + lax.axis_index("c")
        b0 = wid * rows_per_w
        iota = lax.iota(jnp.int32, 16)

        def extract(b, sup_b, out_b):
            for r0 in GROUPS:
                o_vec = ofs_v[b, pl.ds(r0, 16)]
                for l in range(16):
                    r = r0 + l
                    rows = jnp.full((16,), r, jnp.int32)
                    c0 = o_vec[l] + iota
                    v0 = plsc.load_gather(sup_b, [rows, c0])
                    v1 = plsc.load_gather(sup_b, [rows, c0 + 16])
                    out_b[r, pl.ds(0, 16)] = v0
                    out_b[r, pl.ds(16, 16)] = v1

        for s in range(n_segs):
            sb = b0 + s * SEG
            pltpu.sync_copy(sidx_hbm.at[pl.ds(sb, SEG)], sidx_v)
            pltpu.sync_copy(ofs_hbm.at[pl.ds(sb, SEG)], ofs_v)

            def group(i, carry):
                for p in range(NB):
                    b = i * NB + p

                    @pl.when(i > 0)
                    def _drain():
                        pltpu.make_async_copy(
                            out_bufs[p], out_hbm.at[sb + b - NB], ssems[p]
                        ).wait()

                    pltpu.async_copy(
                        table_hbm.at[sidx_v.at[b]], sup_bufs[p], gsems[p]
                    )
                for p in range(NB):
                    b = i * NB + p
                    pltpu.make_async_copy(
                        table_hbm.at[sidx_v.at[b]], sup_bufs[p], gsems[p]
                    ).wait()
                    extract(b, sup_bufs[p], out_bufs[p])
                    pltpu.async_copy(
                        out_bufs[p], out_hbm.at[sb + b], ssems[p]
                    )
                return carry

            lax.fori_loop(0, n_groups, group, 0)
            for p in range(NB):
                b = (n_groups - 1) * NB + p
                pltpu.make_async_copy(
                    out_bufs[p], out_hbm.at[sb + b], ssems[p]
                ).wait()

    return body


def kernel(token_ids, emb):
    bsz, hist = token_ids.shape
    d = emb.shape[1]
    per_sup = 128 // d
    n_sup = emb.shape[0] // per_sup
    sup = emb.reshape(n_sup, 4 * d)
    ids = token_ids.astype(jnp.int32)
    sidx = ids // per_sup
    ofs = (ids % per_sup) * d
    return _build(bsz, hist, d, n_sup)(sidx, ofs, sup)


# speedup vs baseline: 1.2402x; 1.2402x over previous
import functools

import jax
import jax.numpy as jnp
from jax import lax
from jax.experimental import pallas as pl
from jax.experimental.pallas import tpu as pltpu
from jax.experimental.pallas import tpu_sc as plsc

NC = 2
NS = 16
NW = NC * NS
NB = 8


@functools.lru_cache(maxsize=None)
def _build(bsz, hist, d):
    assert bsz % (NW * NB) == 0
    rows_per_w = bsz // NW
    n_groups = rows_per_w // NB
    mesh = plsc.VectorSubcoreMesh(core_axis_name="c", subcore_axis_name="s")

    @functools.partial(
        pl.kernel,
        mesh=mesh,
        compiler_params=pltpu.CompilerParams(
            needs_layout_passes=False, use_tc_tiling_on_sc=False
        ),
        out_type=jax.ShapeDtypeStruct((bsz, hist, d), jnp.float32),
        scratch_types=[
            pltpu.VMEM((rows_per_w, hist), jnp.int32),
            *[pltpu.VMEM((hist, d), jnp.float32) for _ in range(NB)],
            *[pltpu.SemaphoreType.DMA for _ in range(2 * NB)],
        ],
    )
    def body(idx_hbm, table_hbm, out_hbm, idx_v, *bufs_and_sems):
        bufs = bufs_and_sems[:NB]
        gsems = bufs_and_sems[NB:2 * NB]
        ssems = bufs_and_sems[2 * NB:]
        wid = lax.axis_index("s") * NC + lax.axis_index("c")
        b0 = wid * rows_per_w
        pltpu.sync_copy(idx_hbm.at[pl.ds(b0, rows_per_w)], idx_v)

        def group(i, carry):
            for p in range(NB):
                b = i * NB + p

                @pl.when(i > 0)
                def _drain():
                    pltpu.make_async_copy(
                        bufs[p], out_hbm.at[b0 + b - NB], ssems[p]
                    ).wait()

                pltpu.async_copy(
                    table_hbm.at[idx_v.at[b]], bufs[p], gsems[p]
                )
            for p in range(NB):
                b = i * NB + p
                pltpu.make_async_copy(
                    table_hbm.at[idx_v.at[b]], bufs[p], gsems[p]
                ).wait()
                pltpu.async_copy(bufs[p], out_hbm.at[b0 + b], ssems[p])
            return carry

        lax.fori_loop(0, n_groups, group, 0)
        for p in range(NB):
            b = (n_groups - 1) * NB + p
            pltpu.make_async_copy(
                bufs[p], out_hbm.at[b0 + b], ssems[p]
            ).wait()

    return body


def kernel(token_ids, emb):
    bsz, hist = token_ids.shape
    d = emb.shape[1]
    return _build(bsz, hist, d)(token_ids.astype(jnp.int32), emb)
